# baseline (speedup 1.0000x reference)
import functools

import jax
import jax.numpy as jnp
from jax import lax
from jax.experimental import pallas as pl
from jax.experimental.pallas import tpu as pltpu
from jax.experimental.pallas import tpu_sc as plsc

B = 16384
EDIM = 32


@functools.cache
def _make_sc_gather():
    info = plsc.get_sparse_core_info()
    num_cores, num_subcores = info.num_cores, info.num_subcores
    nw = num_cores * num_subcores
    b_per_w = B // nw

    mesh = plsc.VectorSubcoreMesh(core_axis_name="c", subcore_axis_name="s")

    @functools.partial(
        pl.kernel,
        mesh=mesh,
        out_type=[
            jax.ShapeDtypeStruct((B, EDIM), jnp.float32),
            jax.ShapeDtypeStruct((B, EDIM), jnp.float32),
        ],
        scratch_types=[
            pltpu.VMEM((b_per_w,), jnp.int32),
            pltpu.VMEM((b_per_w, EDIM), jnp.float32),
            pltpu.VMEM((b_per_w,), jnp.int32),
            pltpu.VMEM((b_per_w, EDIM), jnp.float32),
            pltpu.SemaphoreType.DMA,
            pltpu.SemaphoreType.DMA,
        ],
        compiler_params=pltpu.CompilerParams(use_tc_tiling_on_sc=False),
    )
    def sc_gather(utab_hbm, uidx_hbm, mtab_hbm, midx_hbm,
                  uout_hbm, mout_hbm,
                  uidx_v, urows_v, midx_v, mrows_v, usem, msem):
        wid = lax.axis_index("s") * num_cores + lax.axis_index("c")
        base = wid * b_per_w
        pltpu.sync_copy(uidx_hbm.at[pl.ds(base, b_per_w)], uidx_v)
        pltpu.sync_copy(midx_hbm.at[pl.ds(base, b_per_w)], midx_v)
        cu = pltpu.async_copy(utab_hbm.at[uidx_v], urows_v, usem)
        cm = pltpu.async_copy(mtab_hbm.at[midx_v], mrows_v, msem)
        cu.wait()
        cm.wait()
        pltpu.sync_copy(urows_v, uout_hbm.at[pl.ds(base, b_per_w)])
        pltpu.sync_copy(mrows_v, mout_hbm.at[pl.ds(base, b_per_w)])

    return sc_gather


def _mlp_body(ue, me, dn, w1u, w1m, w1d, b1, w2, b2, w3, b3, out):
    h = jnp.dot(ue[...], w1u[...], preferred_element_type=jnp.float32)
    h = h + jnp.dot(me[...], w1m[...], preferred_element_type=jnp.float32)
    h = h + jnp.dot(dn[...], w1d[...], preferred_element_type=jnp.float32)
    h = jnp.maximum(h + b1[...], 0.0)
    h = jnp.dot(h, w2[...], preferred_element_type=jnp.float32)
    h = jnp.maximum(h + b2[...], 0.0)
    o = jnp.dot(h, w3[...], preferred_element_type=jnp.float32) + b3[...]
    out[...] = 6.0 * jax.nn.sigmoid(o)


def _mlp(ue, me, dn, w1u, w1m, w1d, b1, w2, b2, w3, b3, blk=2048):
    grid = B // blk
    h1 = w1u.shape[1]
    h2 = w2.shape[1]
    ddim = dn.shape[1]

    def row_spec(d):
        return pl.BlockSpec((blk, d), lambda i: (i, 0))

    def rep_spec(shape):
        nd = len(shape)
        return pl.BlockSpec(shape, lambda i: (0,) * nd)

    return pl.pallas_call(
        _mlp_body,
        grid=(grid,),
        in_specs=[
            row_spec(EDIM),
            row_spec(EDIM),
            row_spec(ddim),
            rep_spec((EDIM, h1)),
            rep_spec((EDIM, h1)),
            rep_spec((ddim, h1)),
            rep_spec((h1,)),
            rep_spec((h1, h2)),
            rep_spec((h2,)),
            rep_spec((h2, 1)),
            rep_spec((1,)),
        ],
        out_specs=pl.BlockSpec((blk, 1), lambda i: (i, 0)),
        out_shape=jax.ShapeDtypeStruct((B, 1), jnp.float32),
    )(ue, me, dn, w1u, w1m, w1d, b1, w2, b2, w3, b3)


def kernel(users, genders, ages, movies, genres, user_table, movie_table,
           W1, b1, W2, b2, W3, b3):
    users = users.astype(jnp.int32)
    movies = movies.astype(jnp.int32)
    ue, me = _make_sc_gather()(user_table, users, movie_table, movies)
    dense = jnp.concatenate([genders, ages, genres], axis=1)
    w1u = W1[:32]
    w1d = jnp.concatenate([W1[32:41], W1[73:91]], axis=0)
    w1m = W1[41:73]
    return _mlp(ue, me, dense, w1u, w1m, w1d, b1, W2, b2, W3, b3)
